# baseline (speedup 1.0000x reference)
import functools

import jax
import jax.numpy as jnp
from jax import lax
from jax.experimental import pallas as pl
from jax.experimental.pallas import tpu as pltpu
from jax.experimental.pallas import tpu_sc as plsc

_B = 4
_S = 4096
_H = 1024
_EPS = 1e-12
_NC = 2
_NS = 16
_NW = _NC * _NS
_SPW = _S // _NW
_C = 16
_NCHUNK = _SPW // _C
_LANES = _H // 16


def _vsum(x):
    iota = lax.iota(jnp.int32, 16)
    for k in (8, 4, 2, 1):
        x = x + jnp.take_along_axis(x, iota ^ k, axis=0)
    return x


def _rsqrt(x):
    xi = lax.bitcast_convert_type(x, jnp.int32)
    yi = jnp.int32(0x5F3759DF) - lax.shift_right_logical(xi, 1)
    y = lax.bitcast_convert_type(yi, jnp.float32)
    for _ in range(3):
        y = y * (1.5 - 0.5 * x * y * y)
    return y


def _body(ids_hbm, tok_hbm, pos_hbm, gam_hbm, bet_hbm, out_hbm,
          ids_v, pos_v, tok_v, gam_v, bet_v, sem):
    wid = lax.axis_index("s") * _NC + lax.axis_index("c")
    s0 = wid * _SPW

    pltpu.sync_copy(gam_hbm, gam_v)
    pltpu.sync_copy(bet_hbm, bet_v)
    for b in range(_B):
        pltpu.sync_copy(ids_hbm.at[pl.ds(b * _S + s0, _SPW)],
                        ids_v.at[pl.ds(b * _SPW, _SPW)])

    @pl.loop(0, _NCHUNK)
    def _chunk(ci):
        spos = s0 + ci * _C
        pltpu.sync_copy(pos_hbm.at[pl.ds(spos, _C)], pos_v)

        for b in range(_B):
            idx = ids_v[pl.ds(b * _SPW + ci * _C, _C)]
            pltpu.async_copy(tok_hbm.at[idx], tok_v, sem).wait()

            @pl.loop(0, _C)
            def _row(r):
                def acc_body(i, carry):
                    acc, acc2 = carry
                    sl = pl.ds(i * 16, 16)
                    x = tok_v[r, sl] + pos_v[r, sl]
                    tok_v[r, sl] = x
                    return acc + x, acc2 + x * x

                zero = jnp.zeros((16,), jnp.float32)
                acc, acc2 = lax.fori_loop(0, _LANES, acc_body, (zero, zero))
                inv_n = jnp.float32(1.0 / _H)
                mean = _vsum(acc) * inv_n
                var = _vsum(acc2) * inv_n - mean * mean
                rstd = _rsqrt(var + jnp.float32(_EPS))

                def norm_body(i, _):
                    sl = pl.ds(i * 16, 16)
                    x = tok_v[r, sl]
                    tok_v[r, sl] = (x - mean) * rstd * gam_v[sl] + bet_v[sl]
                    return 0

                lax.fori_loop(0, _LANES, norm_body, 0)

            pltpu.sync_copy(tok_v, out_hbm.at[pl.ds(b * _S + spos, _C)])


@jax.jit
def _emb(ids, tok, pos, gamma, beta):
    mesh = plsc.VectorSubcoreMesh(core_axis_name="c", subcore_axis_name="s",
                                  num_cores=_NC, num_subcores=_NS)
    f = pl.kernel(
        _body,
        out_type=jax.ShapeDtypeStruct((_B * _S, _H), jnp.float32),
        mesh=mesh,
        scratch_types=[
            pltpu.VMEM((_B * _SPW,), jnp.int32),
            pltpu.VMEM((_C, _H), jnp.float32),
            pltpu.VMEM((_C, _H), jnp.float32),
            pltpu.VMEM((_H,), jnp.float32),
            pltpu.VMEM((_H,), jnp.float32),
            pltpu.SemaphoreType.DMA,
        ],
    )
    return f(ids, tok, pos, gamma, beta)


def kernel(input_ids, token_table, pos_table, gamma, beta):
    ids = input_ids.reshape(-1).astype(jnp.int32)
    out = _emb(ids, token_table, pos_table, gamma, beta)
    return out.reshape(_B, _S, _H)


# speedup vs baseline: 1.2171x; 1.2171x over previous
import functools

import jax
import jax.numpy as jnp
from jax import lax
from jax.experimental import pallas as pl
from jax.experimental.pallas import tpu as pltpu
from jax.experimental.pallas import tpu_sc as plsc

_B = 4
_S = 4096
_H = 1024
_EPS = 1e-12
_NC = 2
_NS = 16
_NW = _NC * _NS
_SPW = _S // _NW
_C = 16
_NCHUNK = _SPW // _C
_LANES = _H // 16


def _vsum(x):
    iota = lax.iota(jnp.int32, 16)
    for k in (8, 4, 2, 1):
        x = x + jnp.take_along_axis(x, iota ^ k, axis=0)
    return x


def _rsqrt(x):
    xi = lax.bitcast_convert_type(x, jnp.int32)
    yi = jnp.int32(0x5F3759DF) - lax.shift_right_logical(xi, 1)
    y = lax.bitcast_convert_type(yi, jnp.float32)
    for _ in range(3):
        y = y * (1.5 - 0.5 * x * y * y)
    return y


def _body(ids_hbm, tok_hbm, pos_hbm, gam_hbm, bet_hbm, out_hbm,
          ids_v, pos_v, tok_v, gam_v, bet_v, gsems, psems):
    wid = lax.axis_index("s") * _NC + lax.axis_index("c")
    s0 = wid * _SPW

    pltpu.sync_copy(gam_hbm, gam_v)
    pltpu.sync_copy(bet_hbm, bet_v)
    for b in range(_B):
        pltpu.sync_copy(ids_hbm.at[pl.ds(b * _S + s0, _SPW)],
                        ids_v.at[pl.ds(b * _SPW, _SPW)])

    def issue_gather(off, z):
        idx = ids_v[pl.ds(off, _C)]
        return pltpu.async_copy(tok_hbm.at[idx], tok_v.at[z], gsems.at[z])

    def issue_pos(spos, z):
        return pltpu.async_copy(pos_hbm.at[pl.ds(spos, _C)], pos_v.at[z],
                                psems.at[z])

    def wait(sem, dst):
        pltpu.make_async_copy(out_hbm.at[pl.ds(0, _C)], dst, sem).wait()

    issue_gather(0, 0)
    issue_pos(s0, 0)

    @pl.loop(0, _NCHUNK, step=2)
    def _chunk(cil):
        for pi in range(2):
            ci = cil + pi
            for b in range(_B):
                z = b % 2
                zn = (z + 1) % 2
                if b == 0:
                    wait(psems.at[pi], pos_v.at[pi])

                    @pl.when(ci < _NCHUNK - 1)
                    def _():
                        issue_pos(s0 + (ci + 1) * _C, (pi + 1) % 2)

                if b < _B - 1:
                    issue_gather((b + 1) * _SPW + ci * _C, zn)
                else:
                    @pl.when(ci < _NCHUNK - 1)
                    def _():
                        issue_gather((ci + 1) * _C, zn)

                wait(gsems.at[z], tok_v.at[z])

                @pl.loop(0, _C)
                def _row(r):
                    def acc_body(i, carry):
                        acc, acc2 = carry
                        sl = pl.ds(i * 16, 16)
                        x = tok_v[z, r, sl] + pos_v[pi, r, sl]
                        tok_v[z, r, sl] = x
                        return acc + x, acc2 + x * x

                    zero = jnp.zeros((16,), jnp.float32)
                    acc, acc2 = lax.fori_loop(0, _LANES, acc_body,
                                              (zero, zero), unroll=4)
                    inv_n = jnp.float32(1.0 / _H)
                    mean = _vsum(acc) * inv_n
                    var = _vsum(acc2) * inv_n - mean * mean
                    rstd = _rsqrt(var + jnp.float32(_EPS))

                    def norm_body(i, _):
                        sl = pl.ds(i * 16, 16)
                        x = tok_v[z, r, sl]
                        tok_v[z, r, sl] = ((x - mean) * rstd * gam_v[sl]
                                           + bet_v[sl])
                        return 0

                    lax.fori_loop(0, _LANES, norm_body, 0, unroll=4)

                pltpu.sync_copy(tok_v.at[z],
                                out_hbm.at[pl.ds(b * _S + s0 + ci * _C, _C)])


@jax.jit
def _emb(ids, tok, pos, gamma, beta):
    mesh = plsc.VectorSubcoreMesh(core_axis_name="c", subcore_axis_name="s",
                                  num_cores=_NC, num_subcores=_NS)
    f = pl.kernel(
        _body,
        out_type=jax.ShapeDtypeStruct((_B * _S, _H), jnp.float32),
        mesh=mesh,
        scratch_types=[
            pltpu.VMEM((_B * _SPW,), jnp.int32),
            pltpu.VMEM((2, _C, _H), jnp.float32),
            pltpu.VMEM((2, _C, _H), jnp.float32),
            pltpu.VMEM((_H,), jnp.float32),
            pltpu.VMEM((_H,), jnp.float32),
            pltpu.SemaphoreType.DMA((2,)),
            pltpu.SemaphoreType.DMA((2,)),
        ],
    )
    return f(ids, tok, pos, gamma, beta)


def kernel(input_ids, token_table, pos_table, gamma, beta):
    ids = input_ids.reshape(-1).astype(jnp.int32)
    out = _emb(ids, token_table, pos_table, gamma, beta)
    return out.reshape(_B, _S, _H)
